# baseline (speedup 1.0000x reference)
import functools

import jax
import jax.numpy as jnp
from jax import lax
from jax.experimental import pallas as pl
from jax.experimental.pallas import tpu as pltpu
from jax.experimental.pallas import tpu_sc as plsc

N = 4096
DIM = 1024
E = 8
HID = 1536

BT = 128
NB = N // BT + E
NPAD = NB * BT

GATE_BG = 512
GATE_NBLK = N // GATE_BG

NC = 2
NS = 16
NW = NC * NS
ROWS_PER_CHUNK = 64


def _gate_body(x_ref, wg_ref, bg_ref, top1_ref, rank_ref, counts_ref,
               base_ref):
    b = pl.program_id(0)

    @pl.when(b == 0)
    def _():
        base_ref[...] = jnp.zeros((1, E), jnp.float32)

    logits = lax.dot(x_ref[...], wg_ref[...],
                     preferred_element_type=jnp.float32) + bg_ref[...]
    m = jnp.max(logits, axis=1, keepdims=True)
    ie = lax.broadcasted_iota(jnp.int32, (GATE_BG, E), 1)
    top1 = jnp.min(jnp.where(logits == m, ie, E), axis=1)
    onehot = (ie == top1[:, None]).astype(jnp.float32)
    tril = (lax.broadcasted_iota(jnp.int32, (GATE_BG, GATE_BG), 0)
            >= lax.broadcasted_iota(jnp.int32, (GATE_BG, GATE_BG), 1)
            ).astype(jnp.float32)
    incl = lax.dot(tril, onehot, preferred_element_type=jnp.float32)
    base = base_ref[...]
    rank = jnp.sum(onehot * (incl + base), axis=1) - 1.0
    top1_ref[...] = top1
    rank_ref[...] = rank.astype(jnp.int32)
    newbase = base + jnp.sum(onehot, axis=0, keepdims=True)
    base_ref[...] = newbase

    @pl.when(b == GATE_NBLK - 1)
    def _():
        counts_ref[...] = newbase.astype(jnp.int32)


def _gate(x, Wg, bg):
    return pl.pallas_call(
        _gate_body,
        grid=(GATE_NBLK,),
        in_specs=[
            pl.BlockSpec((GATE_BG, DIM), lambda b: (b, 0)),
            pl.BlockSpec((DIM, E), lambda b: (0, 0)),
            pl.BlockSpec((1, E), lambda b: (0, 0)),
        ],
        out_specs=[
            pl.BlockSpec((GATE_BG,), lambda b: (b,)),
            pl.BlockSpec((GATE_BG,), lambda b: (b,)),
            pl.BlockSpec((1, E), lambda b: (0, 0)),
        ],
        out_shape=[
            jax.ShapeDtypeStruct((N,), jnp.int32),
            jax.ShapeDtypeStruct((N,), jnp.int32),
            jax.ShapeDtypeStruct((1, E), jnp.int32),
        ],
        scratch_shapes=[pltpu.VMEM((1, E), jnp.float32)],
        compiler_params=pltpu.CompilerParams(
            dimension_semantics=("arbitrary",)),
    )(x, Wg, bg.reshape(1, E))


def _gmm_body(be_ref, xs_ref, w1_ref, b1_ref, w2_ref, b2_ref, out_ref,
              w1b_ref, w2b_ref):
    b = pl.program_id(0)
    changed = jnp.logical_or(b == 0,
                             be_ref[b] != be_ref[jnp.maximum(b - 1, 0)])

    @pl.when(changed)
    def _():
        w1b_ref[...] = w1_ref[0].astype(jnp.bfloat16)
        w2b_ref[...] = w2_ref[0].astype(jnp.bfloat16)

    xb = xs_ref[...]
    h = jnp.maximum(
        lax.dot(xb.astype(jnp.bfloat16), w1b_ref[...],
                preferred_element_type=jnp.float32) + b1_ref[0], 0.0)
    out_ref[...] = (
        lax.dot(h.astype(jnp.bfloat16), w2b_ref[...],
                preferred_element_type=jnp.float32)
        + b2_ref[0] + xb)


def _gmm(block_expert, xs, W1, b1, W2, b2):
    grid_spec = pltpu.PrefetchScalarGridSpec(
        num_scalar_prefetch=1,
        grid=(NB,),
        in_specs=[
            pl.BlockSpec((BT, DIM), lambda b, be: (b, 0)),
            pl.BlockSpec((1, DIM, HID), lambda b, be: (be[b], 0, 0)),
            pl.BlockSpec((1, 1, HID), lambda b, be: (be[b], 0, 0)),
            pl.BlockSpec((1, HID, DIM), lambda b, be: (be[b], 0, 0)),
            pl.BlockSpec((1, 1, DIM), lambda b, be: (be[b], 0, 0)),
        ],
        out_specs=pl.BlockSpec((BT, DIM), lambda b, be: (b, 0)),
        scratch_shapes=[
            pltpu.VMEM((DIM, HID), jnp.bfloat16),
            pltpu.VMEM((HID, DIM), jnp.bfloat16),
        ],
    )
    return pl.pallas_call(
        _gmm_body,
        grid_spec=grid_spec,
        out_shape=jax.ShapeDtypeStruct((NPAD, DIM), jnp.float32),
        compiler_params=pltpu.CompilerParams(
            dimension_semantics=("arbitrary",)),
    )(block_expert, xs, W1, b1.reshape(E, 1, HID), W2, b2.reshape(E, 1, DIM))


@functools.cache
def _sc_mesh():
    return plsc.VectorSubcoreMesh(core_axis_name="c", subcore_axis_name="s",
                                  num_cores=NC)


def _worker_base():
    wid = lax.axis_index("s") * NC + lax.axis_index("c")
    return wid * (N // NW)


def _sc_scatter_rows(x, pos):
    @functools.partial(
        pl.kernel,
        mesh=_sc_mesh(),
        out_type=jax.ShapeDtypeStruct((NPAD, DIM), jnp.float32),
        scratch_types=[
            pltpu.VMEM((ROWS_PER_CHUNK,), jnp.int32),
            pltpu.VMEM((ROWS_PER_CHUNK, DIM), jnp.float32),
            pltpu.SemaphoreType.DMA,
        ],
    )
    def k(x_hbm, pos_hbm, xs_hbm, idx_v, rows_v, sem):
        base = _worker_base()
        for c in range(N // NW // ROWS_PER_CHUNK):
            off = base + c * ROWS_PER_CHUNK
            pltpu.sync_copy(pos_hbm.at[pl.ds(off, ROWS_PER_CHUNK)], idx_v)
            pltpu.sync_copy(x_hbm.at[pl.ds(off, ROWS_PER_CHUNK)], rows_v)
            pltpu.async_copy(rows_v, xs_hbm.at[idx_v], sem).wait()

    return k(x, pos)


def _sc_gather_rows(ys, pos):
    @functools.partial(
        pl.kernel,
        mesh=_sc_mesh(),
        out_type=jax.ShapeDtypeStruct((N, DIM), jnp.float32),
        scratch_types=[
            pltpu.VMEM((ROWS_PER_CHUNK,), jnp.int32),
            pltpu.VMEM((ROWS_PER_CHUNK, DIM), jnp.float32),
            pltpu.SemaphoreType.DMA,
        ],
    )
    def k(ys_hbm, pos_hbm, out_hbm, idx_v, rows_v, sem):
        base = _worker_base()
        for c in range(N // NW // ROWS_PER_CHUNK):
            off = base + c * ROWS_PER_CHUNK
            pltpu.sync_copy(pos_hbm.at[pl.ds(off, ROWS_PER_CHUNK)], idx_v)
            pltpu.async_copy(ys_hbm.at[idx_v], rows_v, sem).wait()
            pltpu.sync_copy(rows_v, out_hbm.at[pl.ds(off, ROWS_PER_CHUNK)])

    return k(ys, pos)


def kernel(x, Wg, bg, W1, b1, W2, b2):
    top1, rank, counts2d = _gate(x, Wg, bg)
    counts = counts2d[0]

    blocks_per_e = (counts + (BT - 1)) // BT
    cumb = jnp.cumsum(blocks_per_e)
    starts = (cumb - blocks_per_e) * BT
    pos = starts[top1] + rank

    bidx = jnp.arange(NB, dtype=jnp.int32)
    be = jnp.sum((bidx[:, None] >= cumb[None, :]).astype(jnp.int32), axis=1)
    used = cumb[-1]
    block_expert = jnp.where(bidx < used, be, be[jnp.maximum(used - 1, 0)])

    xs = _sc_scatter_rows(x, pos)
    ys = _gmm(block_expert, xs, W1, b1, W2, b2)
    return _sc_gather_rows(ys, pos)


# speedup vs baseline: 1.0148x; 1.0148x over previous
import functools

import jax
import jax.numpy as jnp
from jax import lax
from jax.experimental import pallas as pl
from jax.experimental.pallas import tpu as pltpu
from jax.experimental.pallas import tpu_sc as plsc

N = 4096
DIM = 1024
E = 8
HID = 1536

BT = 128
NB = N // BT + E
NPAD = NB * BT

GATE_BG = 512
GATE_NBLK = N // GATE_BG

NC = 2
NS = 16
NW = NC * NS
ROWS_PER_CHUNK = 32


def _gate_body(x_ref, wg_ref, bg_ref, top1_ref, rank_ref, counts_ref,
               base_ref):
    b = pl.program_id(0)

    @pl.when(b == 0)
    def _():
        base_ref[...] = jnp.zeros((1, E), jnp.float32)

    logits = lax.dot(x_ref[...], wg_ref[...],
                     preferred_element_type=jnp.float32) + bg_ref[...]
    m = jnp.max(logits, axis=1, keepdims=True)
    ie = lax.broadcasted_iota(jnp.int32, (GATE_BG, E), 1)
    top1 = jnp.min(jnp.where(logits == m, ie, E), axis=1)
    onehot = (ie == top1[:, None]).astype(jnp.float32)
    tril = (lax.broadcasted_iota(jnp.int32, (GATE_BG, GATE_BG), 0)
            >= lax.broadcasted_iota(jnp.int32, (GATE_BG, GATE_BG), 1)
            ).astype(jnp.float32)
    incl = lax.dot(tril, onehot, preferred_element_type=jnp.float32)
    base = base_ref[...]
    rank = jnp.sum(onehot * (incl + base), axis=1) - 1.0
    top1_ref[...] = top1
    rank_ref[...] = rank.astype(jnp.int32)
    newbase = base + jnp.sum(onehot, axis=0, keepdims=True)
    base_ref[...] = newbase

    @pl.when(b == GATE_NBLK - 1)
    def _():
        counts_ref[...] = newbase.astype(jnp.int32)


def _gate(x, Wg, bg):
    return pl.pallas_call(
        _gate_body,
        grid=(GATE_NBLK,),
        in_specs=[
            pl.BlockSpec((GATE_BG, DIM), lambda b: (b, 0)),
            pl.BlockSpec((DIM, E), lambda b: (0, 0)),
            pl.BlockSpec((1, E), lambda b: (0, 0)),
        ],
        out_specs=[
            pl.BlockSpec((GATE_BG,), lambda b: (b,)),
            pl.BlockSpec((GATE_BG,), lambda b: (b,)),
            pl.BlockSpec((1, E), lambda b: (0, 0)),
        ],
        out_shape=[
            jax.ShapeDtypeStruct((N,), jnp.int32),
            jax.ShapeDtypeStruct((N,), jnp.int32),
            jax.ShapeDtypeStruct((1, E), jnp.int32),
        ],
        scratch_shapes=[pltpu.VMEM((1, E), jnp.float32)],
        compiler_params=pltpu.CompilerParams(
            dimension_semantics=("arbitrary",)),
    )(x, Wg, bg.reshape(1, E))


def _gmm_body(be_ref, xs_ref, w1_ref, b1_ref, w2_ref, b2_ref, out_ref):
    del be_ref
    xb = xs_ref[...]
    h = jnp.maximum(
        lax.dot(xb, w1_ref[0], preferred_element_type=jnp.float32)
        + b1_ref[0], 0.0)
    out_ref[...] = (
        lax.dot(h, w2_ref[0], preferred_element_type=jnp.float32)
        + b2_ref[0] + xb)


def _gmm(block_expert, xs, W1, b1, W2, b2):
    grid_spec = pltpu.PrefetchScalarGridSpec(
        num_scalar_prefetch=1,
        grid=(NB,),
        in_specs=[
            pl.BlockSpec((BT, DIM), lambda b, be: (b, 0)),
            pl.BlockSpec((1, DIM, HID), lambda b, be: (be[b], 0, 0)),
            pl.BlockSpec((1, 1, HID), lambda b, be: (be[b], 0, 0)),
            pl.BlockSpec((1, HID, DIM), lambda b, be: (be[b], 0, 0)),
            pl.BlockSpec((1, 1, DIM), lambda b, be: (be[b], 0, 0)),
        ],
        out_specs=pl.BlockSpec((BT, DIM), lambda b, be: (b, 0)),
    )
    return pl.pallas_call(
        _gmm_body,
        grid_spec=grid_spec,
        out_shape=jax.ShapeDtypeStruct((NPAD, DIM), jnp.float32),
        compiler_params=pltpu.CompilerParams(
            dimension_semantics=("arbitrary",)),
    )(block_expert, xs, W1, b1.reshape(E, 1, HID), W2, b2.reshape(E, 1, DIM))


@functools.cache
def _sc_mesh():
    return plsc.VectorSubcoreMesh(core_axis_name="c", subcore_axis_name="s",
                                  num_cores=NC)


NCHUNK = (N // NW) // ROWS_PER_CHUNK


def _sc_scatter_rows(x, pos):
    @functools.partial(
        pl.kernel,
        mesh=_sc_mesh(),
        out_type=jax.ShapeDtypeStruct((NPAD, DIM), jnp.float32),
        scratch_types=[
            [pltpu.VMEM((ROWS_PER_CHUNK,), jnp.int32)] * NCHUNK,
            pltpu.VMEM((2, ROWS_PER_CHUNK, DIM), jnp.float32),
            pltpu.SemaphoreType.DMA,
            pltpu.SemaphoreType.DMA,
            pltpu.SemaphoreType.DMA,
            pltpu.SemaphoreType.DMA,
            pltpu.SemaphoreType.DMA,
        ],
    )
    def k(x_hbm, pos_hbm, xs_hbm, idx_v, rows_v, r0, r1, s0, s1, psem):
        wid = lax.axis_index("s") * NC + lax.axis_index("c")
        base = wid * (N // NW)
        pcopies = [pltpu.async_copy(
            pos_hbm.at[pl.ds(base + c * ROWS_PER_CHUNK, ROWS_PER_CHUNK)],
            idx_v[c], psem) for c in range(NCHUNK)]
        rsem, ssem = (r0, r1), (s0, s1)
        reads, writes = [None] * NCHUNK, [None] * NCHUNK

        def start_read(c):
            p = c % 2
            reads[c] = pltpu.async_copy(
                x_hbm.at[pl.ds(base + c * ROWS_PER_CHUNK, ROWS_PER_CHUNK)],
                rows_v.at[p], rsem[p])

        start_read(0)
        for c in range(NCHUNK):
            p = c % 2
            reads[c].wait()
            pcopies[c].wait()
            if c + 1 < NCHUNK:
                if c >= 1:
                    writes[c - 1].wait()
                start_read(c + 1)
            writes[c] = pltpu.async_copy(
                rows_v.at[p], xs_hbm.at[idx_v[c]], ssem[p])
        writes[NCHUNK - 2].wait()
        writes[NCHUNK - 1].wait()

    return k(x, pos)


def _sc_gather_rows(ys, pos):
    @functools.partial(
        pl.kernel,
        mesh=_sc_mesh(),
        out_type=jax.ShapeDtypeStruct((N, DIM), jnp.float32),
        scratch_types=[
            [pltpu.VMEM((ROWS_PER_CHUNK,), jnp.int32)] * NCHUNK,
            pltpu.VMEM((2, ROWS_PER_CHUNK, DIM), jnp.float32),
            pltpu.SemaphoreType.DMA,
            pltpu.SemaphoreType.DMA,
            pltpu.SemaphoreType.DMA,
            pltpu.SemaphoreType.DMA,
            pltpu.SemaphoreType.DMA,
        ],
    )
    def k(ys_hbm, pos_hbm, out_hbm, idx_v, rows_v, r0, r1, s0, s1, psem):
        wid = lax.axis_index("s") * NC + lax.axis_index("c")
        base = wid * (N // NW)
        pcopies = [pltpu.async_copy(
            pos_hbm.at[pl.ds(base + c * ROWS_PER_CHUNK, ROWS_PER_CHUNK)],
            idx_v[c], psem) for c in range(NCHUNK)]
        rsem, ssem = (r0, r1), (s0, s1)
        reads, writes = [None] * NCHUNK, [None] * NCHUNK

        def start_read(c):
            p = c % 2
            reads[c] = pltpu.async_copy(
                ys_hbm.at[idx_v[c]], rows_v.at[p], rsem[p])

        pcopies[0].wait()
        start_read(0)
        for c in range(NCHUNK):
            p = c % 2
            if c + 1 < NCHUNK:
                pcopies[c + 1].wait()
            reads[c].wait()
            if c + 1 < NCHUNK:
                if c >= 1:
                    writes[c - 1].wait()
                start_read(c + 1)
            writes[c] = pltpu.async_copy(
                rows_v.at[p],
                out_hbm.at[pl.ds(base + c * ROWS_PER_CHUNK, ROWS_PER_CHUNK)],
                ssem[p])
        writes[NCHUNK - 2].wait()
        writes[NCHUNK - 1].wait()

    return k(ys, pos)


def kernel(x, Wg, bg, W1, b1, W2, b2):
    top1, rank, counts2d = _gate(x, Wg, bg)
    counts = counts2d[0]

    blocks_per_e = (counts + (BT - 1)) // BT
    cumb = jnp.cumsum(blocks_per_e)
    starts = (cumb - blocks_per_e) * BT
    pos = starts[top1] + rank

    bidx = jnp.arange(NB, dtype=jnp.int32)
    be = jnp.sum((bidx[:, None] >= cumb[None, :]).astype(jnp.int32), axis=1)
    used = cumb[-1]
    block_expert = jnp.where(bidx < used, be, be[jnp.maximum(used - 1, 0)])

    xs = _sc_scatter_rows(x, pos)
    ys = _gmm(block_expert, xs, W1, b1, W2, b2)
    return _sc_gather_rows(ys, pos)


# speedup vs baseline: 1.0888x; 1.0730x over previous
import functools

import jax
import jax.numpy as jnp
from jax import lax
from jax.experimental import pallas as pl
from jax.experimental.pallas import tpu as pltpu
from jax.experimental.pallas import tpu_sc as plsc

N = 4096
DIM = 1024
E = 8
HID = 1536

BT = 128
NB = N // BT + E
NB_PAD = 128
NPAD = NB * BT

GATE_BG = 512
GATE_NBLK = N // GATE_BG

NC = 2
NS = 16
NW = NC * NS
ROWS_PER_CHUNK = 32


def _gate_body(x_ref, wg_ref, bg_ref, top1_ref, rank_ref, starts_ref,
               bemap_ref, base_ref):
    b = pl.program_id(0)

    @pl.when(b == 0)
    def _():
        base_ref[...] = jnp.zeros((1, 16), jnp.float32)

    logits = lax.dot(x_ref[...], wg_ref[...],
                     preferred_element_type=jnp.float32) + bg_ref[...]
    m = jnp.max(logits, axis=1, keepdims=True)
    ie = lax.broadcasted_iota(jnp.int32, (GATE_BG, E), 1)
    top1 = jnp.min(jnp.where(logits == m, ie, E), axis=1)
    ie16 = lax.broadcasted_iota(jnp.int32, (GATE_BG, 16), 1)
    onehot = (ie16 == top1[:, None]).astype(jnp.float32)
    tril = (lax.broadcasted_iota(jnp.int32, (GATE_BG, GATE_BG), 0)
            >= lax.broadcasted_iota(jnp.int32, (GATE_BG, GATE_BG), 1)
            ).astype(jnp.float32)
    incl = lax.dot(tril, onehot, preferred_element_type=jnp.float32)
    base = base_ref[...]
    rank = jnp.sum(onehot * (incl + base), axis=1) - 1.0
    top1_ref[...] = top1
    rank_ref[...] = rank.astype(jnp.int32)
    newbase = base + jnp.sum(onehot, axis=0, keepdims=True)
    base_ref[...] = newbase

    @pl.when(b == GATE_NBLK - 1)
    def _():
        counts = newbase
        bpe = jnp.floor((counts + (BT - 1)) * (1.0 / BT))
        t16 = (lax.broadcasted_iota(jnp.int32, (16, 16), 0)
               <= lax.broadcasted_iota(jnp.int32, (16, 16), 1)
               ).astype(jnp.float32)
        cumb = lax.dot(bpe, t16, preferred_element_type=jnp.float32)
        starts_ref[...] = ((cumb - bpe) * BT).astype(jnp.int32)
        cumb_col = jnp.transpose(cumb)
        bemat = (lax.broadcasted_iota(jnp.int32, (16, NB_PAD), 1
                                      ).astype(jnp.float32)
                 >= cumb_col).astype(jnp.int32)
        bemap_ref[...] = jnp.minimum(
            jnp.sum(bemat, axis=0, keepdims=True), E - 1)


def _gate(x, Wg, bg):
    return pl.pallas_call(
        _gate_body,
        grid=(GATE_NBLK,),
        in_specs=[
            pl.BlockSpec((GATE_BG, DIM), lambda b: (b, 0)),
            pl.BlockSpec((DIM, E), lambda b: (0, 0)),
            pl.BlockSpec((1, E), lambda b: (0, 0)),
        ],
        out_specs=[
            pl.BlockSpec((GATE_BG,), lambda b: (b,)),
            pl.BlockSpec((GATE_BG,), lambda b: (b,)),
            pl.BlockSpec((1, 16), lambda b: (0, 0)),
            pl.BlockSpec((1, NB_PAD), lambda b: (0, 0)),
        ],
        out_shape=[
            jax.ShapeDtypeStruct((N,), jnp.int32),
            jax.ShapeDtypeStruct((N,), jnp.int32),
            jax.ShapeDtypeStruct((1, 16), jnp.int32),
            jax.ShapeDtypeStruct((1, NB_PAD), jnp.int32),
        ],
        scratch_shapes=[pltpu.VMEM((1, 16), jnp.float32)],
        compiler_params=pltpu.CompilerParams(
            dimension_semantics=("arbitrary",)),
    )(x, Wg, bg.reshape(1, E))


def _gmm_body(be_ref, xs_ref, w1_ref, b1_ref, w2_ref, b2_ref, out_ref):
    del be_ref
    xb = xs_ref[...]
    h = jnp.maximum(
        lax.dot(xb, w1_ref[0], preferred_element_type=jnp.float32)
        + b1_ref[0], 0.0)
    out_ref[...] = (
        lax.dot(h, w2_ref[0], preferred_element_type=jnp.float32)
        + b2_ref[0] + xb)


def _gmm(block_expert, xs, W1, b1, W2, b2):
    grid_spec = pltpu.PrefetchScalarGridSpec(
        num_scalar_prefetch=1,
        grid=(NB,),
        in_specs=[
            pl.BlockSpec((BT, DIM), lambda b, be: (b, 0)),
            pl.BlockSpec((1, DIM, HID), lambda b, be: (be[b], 0, 0)),
            pl.BlockSpec((1, 1, HID), lambda b, be: (be[b], 0, 0)),
            pl.BlockSpec((1, HID, DIM), lambda b, be: (be[b], 0, 0)),
            pl.BlockSpec((1, 1, DIM), lambda b, be: (be[b], 0, 0)),
        ],
        out_specs=pl.BlockSpec((BT, DIM), lambda b, be: (b, 0)),
    )
    return pl.pallas_call(
        _gmm_body,
        grid_spec=grid_spec,
        out_shape=jax.ShapeDtypeStruct((NPAD, DIM), jnp.float32),
        compiler_params=pltpu.CompilerParams(
            dimension_semantics=("arbitrary",)),
    )(block_expert, xs, W1, b1.reshape(E, 1, HID), W2, b2.reshape(E, 1, DIM))


@functools.cache
def _sc_mesh():
    return plsc.VectorSubcoreMesh(core_axis_name="c", subcore_axis_name="s",
                                  num_cores=NC)


NCHUNK = (N // NW) // ROWS_PER_CHUNK


NRING = 3

_SC_SCRATCH = [
    [pltpu.VMEM((ROWS_PER_CHUNK,), jnp.int32)] * NCHUNK,
    pltpu.VMEM((N // NW,), jnp.int32),
    pltpu.VMEM((N // NW,), jnp.int32),
    pltpu.VMEM((16,), jnp.int32),
    pltpu.VMEM((NRING, ROWS_PER_CHUNK, DIM), jnp.float32),
    [pltpu.SemaphoreType.DMA] * NRING,
    [pltpu.SemaphoreType.DMA] * NRING,
    pltpu.SemaphoreType.DMA,
]


def _compute_pos(base, top1_hbm, rank_hbm, starts_hbm, idx_v, t_v, r_v, s_v,
                 psem):
    copies = [
        pltpu.async_copy(starts_hbm, s_v, psem),
        pltpu.async_copy(top1_hbm.at[pl.ds(base, N // NW)], t_v, psem),
        pltpu.async_copy(rank_hbm.at[pl.ds(base, N // NW)], r_v, psem),
    ]
    for cp in copies:
        cp.wait()
    s_vec = s_v[...]
    for c in range(NCHUNK):
        for g in range(ROWS_PER_CHUNK // 16):
            o = c * ROWS_PER_CHUNK + g * 16
            sv = s_vec.at[t_v[pl.ds(o, 16)]].get(mode="promise_in_bounds")
            idx_v[c][pl.ds(g * 16, 16)] = sv + r_v[pl.ds(o, 16)]


def _sc_scatter_rows(x, top1, rank, starts):
    @functools.partial(
        pl.kernel,
        mesh=_sc_mesh(),
        out_type=jax.ShapeDtypeStruct((NPAD, DIM), jnp.float32),
        scratch_types=_SC_SCRATCH,
    )
    def k(x_hbm, top1_hbm, rank_hbm, starts_hbm, xs_hbm,
          idx_v, t_v, r_v, s_v, rows_v, rsem, ssem, psem):
        wid = lax.axis_index("s") * NC + lax.axis_index("c")
        base = wid * (N // NW)
        reads, writes = [None] * NCHUNK, [None] * NCHUNK

        def start_read(c):
            p = c % NRING
            reads[c] = pltpu.async_copy(
                x_hbm.at[pl.ds(base + c * ROWS_PER_CHUNK, ROWS_PER_CHUNK)],
                rows_v.at[p], rsem[p])

        for c in range(NRING):
            start_read(c)
        _compute_pos(base, top1_hbm, rank_hbm, starts_hbm, idx_v,
                     t_v, r_v, s_v, psem)
        for c in range(NCHUNK):
            p = c % NRING
            reads[c].wait()
            writes[c] = pltpu.async_copy(
                rows_v.at[p], xs_hbm.at[idx_v[c]], ssem[p])
            if c + NRING < NCHUNK:
                writes[c].wait()
                start_read(c + NRING)
        for c in range(max(0, NCHUNK - NRING), NCHUNK):
            writes[c].wait()

    return k(x, top1, rank, starts)


def _sc_gather_rows(ys, top1, rank, starts):
    @functools.partial(
        pl.kernel,
        mesh=_sc_mesh(),
        out_type=jax.ShapeDtypeStruct((N, DIM), jnp.float32),
        scratch_types=_SC_SCRATCH,
    )
    def k(ys_hbm, top1_hbm, rank_hbm, starts_hbm, out_hbm,
          idx_v, t_v, r_v, s_v, rows_v, rsem, ssem, psem):
        wid = lax.axis_index("s") * NC + lax.axis_index("c")
        base = wid * (N // NW)
        reads, writes = [None] * NCHUNK, [None] * NCHUNK

        def start_read(c):
            p = c % NRING
            reads[c] = pltpu.async_copy(
                ys_hbm.at[idx_v[c]], rows_v.at[p], rsem[p])

        _compute_pos(base, top1_hbm, rank_hbm, starts_hbm, idx_v,
                     t_v, r_v, s_v, psem)
        for c in range(NRING):
            start_read(c)
        for c in range(NCHUNK):
            p = c % NRING
            reads[c].wait()
            writes[c] = pltpu.async_copy(
                rows_v.at[p],
                out_hbm.at[pl.ds(base + c * ROWS_PER_CHUNK, ROWS_PER_CHUNK)],
                ssem[p])
            if c + NRING < NCHUNK:
                writes[c].wait()
                start_read(c + NRING)
        for c in range(max(0, NCHUNK - NRING), NCHUNK):
            writes[c].wait()

    return k(ys, top1, rank, starts)


def kernel(x, Wg, bg, W1, b1, W2, b2):
    top1, rank, starts16, bemap = _gate(x, Wg, bg)
    starts = starts16.reshape(16)
    block_expert = bemap.reshape(NB_PAD)

    xs = _sc_scatter_rows(x, top1, rank, starts)
    ys = _gmm(block_expert, xs, W1, b1, W2, b2)
    return _sc_gather_rows(ys, top1, rank, starts)
